# baseline (speedup 1.0000x reference)
import functools

import jax
import jax.numpy as jnp
from jax import lax
from jax.experimental import pallas as pl
from jax.experimental.pallas import tpu as pltpu
import jax.experimental.pallas.tpu_sc as plsc

N = 10000
E = 320000
D = 128
H = 128
FINAL = 384
NPAD = 10240
NC = 2
NS = 16
NW = NC * NS
CHUNK = 128
NCHUNKS = E // CHUNK
ROWS_PER_TILE = NPAD // NS
NFULL = 78


def _deg_body(edge_hbm, zeros_hbm, ones_hbm, out_hbm, idx_v, ones_v, deg_sh,
              isem0, isem1):
    c = lax.axis_index("c")
    s = lax.axis_index("s")
    w = c * NS + s
    nk = (NCHUNKS - w + NW - 1) // NW
    isems = (isem0, isem1)

    def idx_copy(j, b):
        base = (w + j * NW) * CHUNK
        return pltpu.make_async_copy(
            edge_hbm.at[pl.ds(1, 1), pl.ds(base, CHUNK)], idx_v.at[b],
            isems[b])

    idx_copy(0, 0).start()
    pltpu.sync_copy(zeros_hbm,
                    deg_sh.at[pl.ds(s * ROWS_PER_TILE, ROWS_PER_TILE)])
    pltpu.sync_copy(ones_hbm, ones_v)
    plsc.subcore_barrier()

    def group(gidx, _):
        for bb in (0, 1):
            j = gidx * 2 + bb

            @pl.when(j + 1 < nk)
            def _():
                idx_copy(j + 1, 1 - bb).start()

            idx_copy(j, bb).wait()
            pltpu.sync_copy(ones_v, deg_sh.at[idx_v.at[bb, 0]], add=True)
        return 0

    lax.fori_loop(0, NFULL // 2, group, 0)

    @pl.when(nk > NFULL)
    def _():
        idx_copy(NFULL, 0).wait()
        pltpu.sync_copy(ones_v, deg_sh.at[idx_v.at[0, 0]], add=True)

    plsc.subcore_barrier()
    pltpu.sync_copy(deg_sh.at[pl.ds(s * ROWS_PER_TILE, ROWS_PER_TILE)],
                    out_hbm.at[c, pl.ds(s * ROWS_PER_TILE, ROWS_PER_TILE)])


def _deg_call(edge_index):
    mesh = plsc.VectorSubcoreMesh(core_axis_name="c", subcore_axis_name="s")
    f = pl.kernel(
        _deg_body,
        out_type=jax.ShapeDtypeStruct((NC, NPAD), jnp.float32),
        mesh=mesh,
        scratch_types=[
            pltpu.VMEM((2, 1, CHUNK), jnp.int32),
            pltpu.VMEM((CHUNK,), jnp.float32),
            pltpu.VMEM_SHARED((NPAD,), jnp.float32),
            pltpu.SemaphoreType.DMA,
            pltpu.SemaphoreType.DMA,
        ],
    )
    return f(edge_index, jnp.zeros((ROWS_PER_TILE,), jnp.float32),
             jnp.ones((CHUNK,), jnp.float32))


def _edge_body(g_hbm, edge_hbm, zeros_hbm, out_hbm, idx_v, rows_v, agg_sh,
               isem0, isem1, isem2, gsem0, gsem1, ssem0, ssem1):
    c = lax.axis_index("c")
    s = lax.axis_index("s")
    w = c * NS + s
    nk = (NCHUNKS - w + NW - 1) // NW
    isems = (isem0, isem1, isem2)
    gsems = (gsem0, gsem1)
    ssems = (ssem0, ssem1)

    def idx_copy(j, b3):
        base = (w + j * NW) * CHUNK
        return pltpu.make_async_copy(edge_hbm.at[:, pl.ds(base, CHUNK)],
                                     idx_v.at[b3], isems[b3])

    def gather(b3, b2):
        return pltpu.make_async_copy(g_hbm.at[idx_v.at[b3, 0]],
                                     rows_v.at[b2], gsems[b2])

    def scatter_start(b3, b2):
        pltpu.async_copy(rows_v.at[b2], agg_sh.at[idx_v.at[b3, 1]],
                         ssems[b2], add=True)

    def scatter_wait(b3, b2):
        pltpu.make_async_copy(rows_v.at[b2], agg_sh.at[idx_v.at[b3, 1]],
                              ssems[b2]).wait()

    idx_copy(0, 0).start()
    idx_copy(1, 1).start()
    pltpu.sync_copy(zeros_hbm,
                    agg_sh.at[pl.ds(s * ROWS_PER_TILE, ROWS_PER_TILE)])
    idx_copy(0, 0).wait()
    gather(0, 0).start()
    plsc.subcore_barrier()

    def group(gidx, _):
        for u in range(6):
            j6 = gidx * 6 + u
            b3 = u % 3
            b2 = u % 2

            gather(b3, b2).wait()
            scatter_start(b3, b2)

            @pl.when(j6 >= 1)
            def _():
                scatter_wait((u + 2) % 3, (u + 1) % 2)

            @pl.when(j6 + 2 < nk)
            def _():
                idx_copy(j6 + 2, (u + 2) % 3).start()

            @pl.when(j6 + 1 < nk)
            def _():
                idx_copy(j6 + 1, (u + 1) % 3).wait()
                gather((u + 1) % 3, (u + 1) % 2).start()
        return 0

    lax.fori_loop(0, NFULL // 6, group, 0)

    @pl.when(nk > NFULL)
    def _():
        gather(0, 0).wait()
        scatter_start(0, 0)
        scatter_wait(2, 1)
        scatter_wait(0, 0)

    @pl.when(nk == NFULL)
    def _():
        scatter_wait(2, 1)

    plsc.subcore_barrier()
    pltpu.sync_copy(agg_sh.at[pl.ds(s * ROWS_PER_TILE, ROWS_PER_TILE)],
                    out_hbm.at[c, pl.ds(s * ROWS_PER_TILE, ROWS_PER_TILE)])


def _edge_call(g, edge_index, zeros2d):
    mesh = plsc.VectorSubcoreMesh(core_axis_name="c", subcore_axis_name="s")
    f = pl.kernel(
        _edge_body,
        out_type=jax.ShapeDtypeStruct((NC, NPAD, H), jnp.float32),
        mesh=mesh,
        scratch_types=[
            pltpu.VMEM((3, 2, CHUNK), jnp.int32),
            pltpu.VMEM((2, CHUNK, H), jnp.float32),
            pltpu.VMEM_SHARED((NPAD, H), jnp.float32),
            pltpu.SemaphoreType.DMA,
            pltpu.SemaphoreType.DMA,
            pltpu.SemaphoreType.DMA,
            pltpu.SemaphoreType.DMA,
            pltpu.SemaphoreType.DMA,
            pltpu.SemaphoreType.DMA,
            pltpu.SemaphoreType.DMA,
        ],
    )
    return f(g, edge_index, zeros2d)


def _dinv_col(deg_ref):
    d = deg_ref[0] + deg_ref[1] + 1.0
    return lax.rsqrt(d[:N])


def _tc_pre_body(x_ref, w_ref, deg_ref, g_ref):
    dinv = _dinv_col(deg_ref)
    h = jnp.dot(x_ref[...], w_ref[...], preferred_element_type=jnp.float32)
    g_ref[...] = dinv * h


def _tc_pre(x, W0, deg2):
    return pl.pallas_call(
        _tc_pre_body,
        out_shape=jax.ShapeDtypeStruct((N, H), jnp.float32),
    )(x, W0, deg2)


def _postprocess(agg_ref, g_ref, deg_ref, b_ref, al_ref, ga_ref, be_ref):
    dinv = _dinv_col(deg_ref)
    z = dinv * (agg_ref[0, :N] + agg_ref[1, :N] + g_ref[...]) + b_ref[...]
    mean = jnp.mean(z, axis=0, keepdims=True)
    cent = z - al_ref[...] * mean
    var = jnp.mean(cent * cent, axis=0, keepdims=True)
    gn = ga_ref[...] * cent / jnp.sqrt(var + 1e-5) + be_ref[...]
    return jnp.maximum(gn, 0.0), dinv


def _tc_mid_body(has_res, agg_ref, g_ref, deg_ref, b_ref, al_ref, ga_ref,
                 be_ref, w_ref, *rest):
    if has_res:
        (prev_ref, out_ref, gn_ref) = rest
    else:
        (out_ref, gn_ref) = rest
    out, dinv = _postprocess(agg_ref, g_ref, deg_ref, b_ref, al_ref, ga_ref,
                             be_ref)
    out_ref[...] = out
    xn = out + prev_ref[...] if has_res else out
    hn = jnp.dot(xn, w_ref[...], preferred_element_type=jnp.float32)
    gn_ref[...] = dinv * hn


def _tc_mid(agg, g, deg2, b, al, ga, be, Wn, prev):
    args = [agg, g, deg2, b, al, ga, be, Wn]
    if prev is not None:
        args.append(prev)
    return pl.pallas_call(
        functools.partial(_tc_mid_body, prev is not None),
        out_shape=(jax.ShapeDtypeStruct((N, H), jnp.float32),
                   jax.ShapeDtypeStruct((N, H), jnp.float32)),
    )(*args)


def _tc_final_body(agg_ref, g_ref, deg_ref, b_ref, al_ref, ga_ref, be_ref,
                   o0_ref, o1_ref, lw1_ref, lb1_ref, lw2_ref, lb2_ref, y_ref):
    out2, _ = _postprocess(agg_ref, g_ref, deg_ref, b_ref, al_ref, ga_ref,
                           be_ref)
    m = jnp.maximum(out2, jnp.maximum(o0_ref[...], o1_ref[...]))
    t = jnp.dot(m, lw1_ref[...], preferred_element_type=jnp.float32)
    t = jnp.maximum(t + lb1_ref[...], 0.0)
    y = jnp.dot(t, lw2_ref[...], preferred_element_type=jnp.float32)
    y_ref[...] = y + lb2_ref[...]


def _tc_final(agg, g, deg2, b, al, ga, be, o0, o1, lw1, lb1, lw2, lb2):
    return pl.pallas_call(
        _tc_final_body,
        out_shape=jax.ShapeDtypeStruct((N, FINAL), jnp.float32),
    )(agg, g, deg2, b, al, ga, be, o0, o1, lw1, lb1, lw2, lb2)


def kernel(x, edge_index, W0, b0, alpha0, gamma0, beta0, W1, b1, alpha1,
           gamma1, beta1, W2, b2, alpha2, gamma2, beta2, lw1, lb1, lw2, lb2):
    zeros2d = jnp.zeros((ROWS_PER_TILE, H), jnp.float32)
    deg_parts = _deg_call(edge_index)
    deg2 = deg_parts.reshape(NC, NPAD, 1)
    g0 = _tc_pre(x, W0, deg2)
    agg0 = _edge_call(g0, edge_index, zeros2d)
    out0, g1 = _tc_mid(agg0, g0, deg2, b0, alpha0, gamma0, beta0, W1, None)
    agg1 = _edge_call(g1, edge_index, zeros2d)
    out1, g2 = _tc_mid(agg1, g1, deg2, b1, alpha1, gamma1, beta1, W2, out0)
    agg2 = _edge_call(g2, edge_index, zeros2d)
    y = _tc_final(agg2, g2, deg2, b2, alpha2, gamma2, beta2, out0, out1,
                  lw1, lb1, lw2, lb2)
    return y.reshape(N, 3, H)


# speedup vs baseline: 1.1476x; 1.1476x over previous
import functools

import jax
import jax.numpy as jnp
from jax import lax
from jax.experimental import pallas as pl
from jax.experimental.pallas import tpu as pltpu
import jax.experimental.pallas.tpu_sc as plsc

N = 10000
E = 320000
D = 128
H = 128
FINAL = 384
NPAD = 10240
NC = 2
NS = 16
NW = NC * NS
CHUNK = 128
NCHUNKS = E // CHUNK
ROWS_PER_TILE = NPAD // NS
NFULL = 78


def _deg_body(edge_hbm, zeros_hbm, ones_hbm, out_hbm, idx_v, ones_v, deg_sh,
              isem0, isem1):
    c = lax.axis_index("c")
    s = lax.axis_index("s")
    w = c * NS + s
    nk = (NCHUNKS - w + NW - 1) // NW
    isems = (isem0, isem1)

    def idx_copy(j, b):
        base = (w + j * NW) * CHUNK
        return pltpu.make_async_copy(
            edge_hbm.at[pl.ds(1, 1), pl.ds(base, CHUNK)], idx_v.at[b],
            isems[b])

    idx_copy(0, 0).start()
    pltpu.sync_copy(zeros_hbm,
                    deg_sh.at[pl.ds(s * ROWS_PER_TILE, ROWS_PER_TILE)])
    pltpu.sync_copy(ones_hbm, ones_v)
    plsc.subcore_barrier()

    def group(gidx, _):
        for bb in (0, 1):
            j = gidx * 2 + bb

            @pl.when(j + 1 < nk)
            def _():
                idx_copy(j + 1, 1 - bb).start()

            idx_copy(j, bb).wait()
            pltpu.sync_copy(ones_v, deg_sh.at[idx_v.at[bb, 0]], add=True)
        return 0

    lax.fori_loop(0, NFULL // 2, group, 0)

    @pl.when(nk > NFULL)
    def _():
        idx_copy(NFULL, 0).wait()
        pltpu.sync_copy(ones_v, deg_sh.at[idx_v.at[0, 0]], add=True)

    plsc.subcore_barrier()
    pltpu.sync_copy(deg_sh.at[pl.ds(s * ROWS_PER_TILE, ROWS_PER_TILE)],
                    out_hbm.at[c, pl.ds(s * ROWS_PER_TILE, ROWS_PER_TILE)])


def _deg_call(edge_index):
    mesh = plsc.VectorSubcoreMesh(core_axis_name="c", subcore_axis_name="s")
    f = pl.kernel(
        _deg_body,
        out_type=jax.ShapeDtypeStruct((NC, NPAD), jnp.float32),
        mesh=mesh,
        scratch_types=[
            pltpu.VMEM((2, 1, CHUNK), jnp.int32),
            pltpu.VMEM((CHUNK,), jnp.float32),
            pltpu.VMEM_SHARED((NPAD,), jnp.float32),
            pltpu.SemaphoreType.DMA,
            pltpu.SemaphoreType.DMA,
        ],
    )
    return f(edge_index, jnp.zeros((ROWS_PER_TILE,), jnp.float32),
             jnp.ones((CHUNK,), jnp.float32))


def _edge_body(g_hbm, edge_hbm, zeros_hbm, out_hbm, idx_v, rows_v, agg_sh,
               isem0, isem1, isem2, gsem0, gsem1):
    c = lax.axis_index("c")
    s = lax.axis_index("s")
    w = c * NS + s
    nk = (NCHUNKS - w + NW - 1) // NW
    isems = (isem0, isem1, isem2)
    gsems = (gsem0, gsem1)

    def idx_copy(j, b3):
        base = (w + j * NW) * CHUNK
        return pltpu.make_async_copy(edge_hbm.at[:, pl.ds(base, CHUNK)],
                                     idx_v.at[b3], isems[b3])

    def gather(b3, b2):
        return pltpu.make_async_copy(g_hbm.at[idx_v.at[b3, 0]],
                                     rows_v.at[b2], gsems[b2])

    idx_copy(0, 0).start()
    idx_copy(1, 1).start()
    pltpu.sync_copy(zeros_hbm,
                    agg_sh.at[pl.ds(s * ROWS_PER_TILE, ROWS_PER_TILE)])
    idx_copy(0, 0).wait()
    gather(0, 0).start()
    plsc.subcore_barrier()

    def group(gidx, _):
        for u in range(6):
            j6 = gidx * 6 + u
            b3 = u % 3
            b2 = u % 2

            @pl.when(j6 + 2 < nk)
            def _():
                idx_copy(j6 + 2, (u + 2) % 3).start()

            @pl.when(j6 + 1 < nk)
            def _():
                idx_copy(j6 + 1, (u + 1) % 3).wait()
                gather((u + 1) % 3, (u + 1) % 2).start()

            gather(b3, b2).wait()
            pltpu.sync_copy(rows_v.at[b2], agg_sh.at[idx_v.at[b3, 1]],
                            add=True)
        return 0

    lax.fori_loop(0, NFULL // 6, group, 0)

    @pl.when(nk > NFULL)
    def _():
        gather(0, 0).wait()
        pltpu.sync_copy(rows_v.at[0], agg_sh.at[idx_v.at[0, 1]], add=True)

    plsc.subcore_barrier()
    pltpu.sync_copy(agg_sh.at[pl.ds(s * ROWS_PER_TILE, ROWS_PER_TILE)],
                    out_hbm.at[c, pl.ds(s * ROWS_PER_TILE, ROWS_PER_TILE)])


def _edge_call(g, edge_index, zeros2d):
    mesh = plsc.VectorSubcoreMesh(core_axis_name="c", subcore_axis_name="s")
    f = pl.kernel(
        _edge_body,
        out_type=jax.ShapeDtypeStruct((NC, NPAD, H), jnp.float32),
        mesh=mesh,
        scratch_types=[
            pltpu.VMEM((3, 2, CHUNK), jnp.int32),
            pltpu.VMEM((2, CHUNK, H), jnp.float32),
            pltpu.VMEM_SHARED((NPAD, H), jnp.float32),
            pltpu.SemaphoreType.DMA,
            pltpu.SemaphoreType.DMA,
            pltpu.SemaphoreType.DMA,
            pltpu.SemaphoreType.DMA,
            pltpu.SemaphoreType.DMA,
        ],
    )
    return f(g, edge_index, zeros2d)


def _dinv_col(deg_ref):
    d = deg_ref[0] + deg_ref[1] + 1.0
    return lax.rsqrt(d[:N])


def _tc_pre_body(x_ref, w_ref, deg_ref, g_ref):
    dinv = _dinv_col(deg_ref)
    h = jnp.dot(x_ref[...], w_ref[...], preferred_element_type=jnp.float32)
    g_ref[...] = dinv * h


def _tc_pre(x, W0, deg2):
    return pl.pallas_call(
        _tc_pre_body,
        out_shape=jax.ShapeDtypeStruct((N, H), jnp.float32),
    )(x, W0, deg2)


def _postprocess(agg_ref, g_ref, deg_ref, b_ref, al_ref, ga_ref, be_ref):
    dinv = _dinv_col(deg_ref)
    z = dinv * (agg_ref[0, :N] + agg_ref[1, :N] + g_ref[...]) + b_ref[...]
    mean = jnp.mean(z, axis=0, keepdims=True)
    cent = z - al_ref[...] * mean
    var = jnp.mean(cent * cent, axis=0, keepdims=True)
    gn = ga_ref[...] * cent / jnp.sqrt(var + 1e-5) + be_ref[...]
    return jnp.maximum(gn, 0.0), dinv


def _tc_mid_body(has_res, agg_ref, g_ref, deg_ref, b_ref, al_ref, ga_ref,
                 be_ref, w_ref, *rest):
    if has_res:
        (prev_ref, out_ref, gn_ref) = rest
    else:
        (out_ref, gn_ref) = rest
    out, dinv = _postprocess(agg_ref, g_ref, deg_ref, b_ref, al_ref, ga_ref,
                             be_ref)
    out_ref[...] = out
    xn = out + prev_ref[...] if has_res else out
    hn = jnp.dot(xn, w_ref[...], preferred_element_type=jnp.float32)
    gn_ref[...] = dinv * hn


def _tc_mid(agg, g, deg2, b, al, ga, be, Wn, prev):
    args = [agg, g, deg2, b, al, ga, be, Wn]
    if prev is not None:
        args.append(prev)
    return pl.pallas_call(
        functools.partial(_tc_mid_body, prev is not None),
        out_shape=(jax.ShapeDtypeStruct((N, H), jnp.float32),
                   jax.ShapeDtypeStruct((N, H), jnp.float32)),
    )(*args)


def _tc_final_body(agg_ref, g_ref, deg_ref, b_ref, al_ref, ga_ref, be_ref,
                   o0_ref, o1_ref, lw1_ref, lb1_ref, lw2_ref, lb2_ref, y_ref):
    out2, _ = _postprocess(agg_ref, g_ref, deg_ref, b_ref, al_ref, ga_ref,
                           be_ref)
    m = jnp.maximum(out2, jnp.maximum(o0_ref[...], o1_ref[...]))
    t = jnp.dot(m, lw1_ref[...], preferred_element_type=jnp.float32)
    t = jnp.maximum(t + lb1_ref[...], 0.0)
    y = jnp.dot(t, lw2_ref[...], preferred_element_type=jnp.float32)
    y_ref[...] = y + lb2_ref[...]


def _tc_final(agg, g, deg2, b, al, ga, be, o0, o1, lw1, lb1, lw2, lb2):
    return pl.pallas_call(
        _tc_final_body,
        out_shape=jax.ShapeDtypeStruct((N, FINAL), jnp.float32),
    )(agg, g, deg2, b, al, ga, be, o0, o1, lw1, lb1, lw2, lb2)


def kernel(x, edge_index, W0, b0, alpha0, gamma0, beta0, W1, b1, alpha1,
           gamma1, beta1, W2, b2, alpha2, gamma2, beta2, lw1, lb1, lw2, lb2):
    zeros2d = jnp.zeros((ROWS_PER_TILE, H), jnp.float32)
    deg_parts = _deg_call(edge_index)
    deg2 = deg_parts.reshape(NC, NPAD, 1)
    g0 = _tc_pre(x, W0, deg2)
    agg0 = _edge_call(g0, edge_index, zeros2d)
    out0, g1 = _tc_mid(agg0, g0, deg2, b0, alpha0, gamma0, beta0, W1, None)
    agg1 = _edge_call(g1, edge_index, zeros2d)
    out1, g2 = _tc_mid(agg1, g1, deg2, b1, alpha1, gamma1, beta1, W2, out0)
    agg2 = _edge_call(g2, edge_index, zeros2d)
    y = _tc_final(agg2, g2, deg2, b2, alpha2, gamma2, beta2, out0, out1,
                  lw1, lb1, lw2, lb2)
    return y.reshape(N, 3, H)
